# baseline (speedup 1.0000x reference)
import jax
import jax.numpy as jnp
from jax import lax
from jax.experimental import pallas as pl
from jax.experimental.pallas import tpu as pltpu
from jax.experimental.pallas import tpu_sc as plsc

B, H, W, C = 16, 384, 384, 96
PIX = B * H * W
NC, NS, L = 2, 16, 16
NW = NC * NS
PPW = PIX // NW
N = 96
CHUNKS = PPW // N
G = N // L
CB = 8
NCB = N * CB
Wf = float(W)
Hf = float(H)

_DNUMS = lax.GatherDimensionNumbers(
    offset_dims=(), collapsed_slice_dims=(0,), start_index_map=(0,))


def _build_body(im_hbm, nbr_hbm):
    wid = lax.axis_index("c") * NS + lax.axis_index("s")
    r0 = wid * PPW
    short = PPW - (W + 1)
    pltpu.sync_copy(im_hbm.at[pl.ds(r0, PPW), :],
                    nbr_hbm.at[pl.ds(r0, PPW), pl.ds(0, C)])

    @pl.when(wid < NW - 1)
    def _():
        for k, o in enumerate((1, W, W + 1)):
            pltpu.sync_copy(im_hbm.at[pl.ds(r0 + o, PPW), :],
                            nbr_hbm.at[pl.ds(r0, PPW), pl.ds((k + 1) * C, C)])

    @pl.when(wid == NW - 1)
    def _():
        for k, o in enumerate((1, W, W + 1)):
            pltpu.sync_copy(im_hbm.at[pl.ds(r0 + o, short), :],
                            nbr_hbm.at[pl.ds(r0, short), pl.ds((k + 1) * C, C)])


def _splat(v, l):
    idx = jnp.full((L, 1), l, jnp.int32)
    return lax.gather(v, idx, _DNUMS, (1,),
                      mode=lax.GatherScatterMode.PROMISE_IN_BOUNDS)


def _gen_and_fire(c, pix0, base_v, xf_hbm, yf_hbm, nbr_hbm,
                  cbx, cby, idx, ws, gbuf, sem):
    @pl.when(lax.rem(c, CB) == 0)
    def _():
        off = pix0 + c * N
        pltpu.sync_copy(xf_hbm.at[pl.ds(off, NCB)], cbx)
        pltpu.sync_copy(yf_hbm.at[pl.ds(off, NCB)], cby)

    pos = lax.rem(c, CB) * N
    wA, wB, wC, wD = ws
    for g in range(G):
        s = pl.ds(g * L, L)
        Xf = cbx[pl.ds(pos + g * L, L)]
        Yf = cby[pl.ds(pos + g * L, L)]
        Xp = (Xf + 1.0) / 2.0 * Wf
        Yp = (Yf + 1.0) / 2.0 * Hf
        x0t = Xp.astype(jnp.int32)
        x0 = jnp.where(x0t.astype(jnp.float32) > Xp, x0t - 1, x0t)
        y0t = Yp.astype(jnp.int32)
        y0 = jnp.where(y0t.astype(jnp.float32) > Yp, y0t - 1, y0t)
        m = ((Xp >= 0.0) & (Xp < Wf - 1.0)
             & (Yp >= 0.0) & (Yp < Hf - 1.0))
        x0c = jnp.minimum(jnp.maximum(x0, 0), W - 2)
        y0c = jnp.minimum(jnp.maximum(y0, 0), H - 2)
        x0f = x0c.astype(jnp.float32)
        y0f = y0c.astype(jnp.float32)
        x1f = x0f + 1.0
        y1f = y0f + 1.0
        zero = jnp.zeros((L,), jnp.float32)
        wa = jnp.where(m, (x1f - Xp) * (y1f - Yp), zero)
        wb = jnp.where(m, (x1f - Xp) * (Yp - y0f), zero)
        wc = jnp.where(m, (Xp - x0f) * (y1f - Yp), zero)
        wd = jnp.where(m, (Xp - x0f) * (Yp - y0f), zero)
        idx[s] = base_v + y0c * W + x0c
        wA[s] = wa
        wB[s] = wb
        wC[s] = wc
        wD[s] = wd
    pltpu.async_copy(nbr_hbm.at[idx], gbuf, sem)


def _combine(ws, gbuf, ob):
    wA, wB, wC, wD = ws

    def gbody(g, carry):
        s = pl.ds(g * L, L)
        wa16 = wA[s]
        wb16 = wB[s]
        wc16 = wC[s]
        wd16 = wD[s]
        for l in range(L):
            i = g * L + l
            wal = _splat(wa16, l)
            wbl = _splat(wb16, l)
            wcl = _splat(wc16, l)
            wdl = _splat(wd16, l)
            for j in range(C // L):
                av = gbuf[i, pl.ds(j * L, L)]
                cv = gbuf[i, pl.ds(C + j * L, L)]
                bv = gbuf[i, pl.ds(2 * C + j * L, L)]
                dv = gbuf[i, pl.ds(3 * C + j * L, L)]
                o = av * wal + bv * wbl
                o = o + cv * wcl
                o = o + dv * wdl
                ob[pl.ds(i * C + j * L, L)] = o
        return carry

    lax.fori_loop(0, G, gbody, 0)


def _body(nbr_hbm, xf_hbm, yf_hbm, out_hbm,
          cbx, cby,
          idx0, idx1,
          wA0, wB0, wC0, wD0, wA1, wB1, wC1, wD1,
          g0, g1, ob0, ob1,
          sem_g0, sem_g1, sem_o0, sem_o1):
    wid = lax.axis_index("c") * NS + lax.axis_index("s")
    pix0 = wid * PPW
    img_base = (wid // 2) * (H * W)
    base_v = jnp.full((L,), img_base, jnp.int32)

    idxs = (idx0, idx1)
    ws = ((wA0, wB0, wC0, wD0), (wA1, wB1, wC1, wD1))
    gbufs = (g0, g1)
    obs = (ob0, ob1)
    sem_gs = (sem_g0, sem_g1)
    sem_os = (sem_o0, sem_o1)

    _gen_and_fire(0, pix0, base_v, xf_hbm, yf_hbm, nbr_hbm,
                  cbx, cby, idxs[0], ws[0], gbufs[0], sem_gs[0])

    def outer(i2, carry):
        for d in (0, 1):
            c = i2 * 2 + d
            nd = 1 - d

            @pl.when(c + 1 < CHUNKS)
            def _():
                _gen_and_fire(c + 1, pix0, base_v, xf_hbm, yf_hbm, nbr_hbm,
                              cbx, cby, idxs[nd], ws[nd], gbufs[nd],
                              sem_gs[nd])

            pltpu.make_async_copy(nbr_hbm.at[idxs[d]], gbufs[d],
                                  sem_gs[d]).wait()

            @pl.when(c >= 2)
            def _():
                prev = (pix0 + (c - 2) * N) * C
                pltpu.make_async_copy(
                    obs[d], out_hbm.at[pl.ds(prev, N * C)], sem_os[d]).wait()

            _combine(ws[d], gbufs[d], obs[d])
            cur = (pix0 + c * N) * C
            pltpu.async_copy(obs[d], out_hbm.at[pl.ds(cur, N * C)], sem_os[d])
        return carry

    lax.fori_loop(0, CHUNKS // 2, outer, 0)

    for d in (0, 1):
        last = (pix0 + (CHUNKS - 2 + d) * N) * C
        pltpu.make_async_copy(
            obs[d], out_hbm.at[pl.ds(last, N * C)], sem_os[d]).wait()


@jax.jit
def kernel(im, thetas):
    X, Y = jnp.meshgrid(jnp.linspace(-1.0, 1.0, W), jnp.linspace(-1.0, 1.0, H))
    flat_coords = jnp.concatenate(
        [X.reshape(1, -1), Y.reshape(1, -1),
         jnp.ones((1, H * W), dtype=jnp.float32)], axis=0)
    th = thetas.reshape(-1, 2, 3)
    new_flat = jnp.matmul(th, jnp.broadcast_to(flat_coords[None, :, :],
                                               (B, 3, H * W)))
    Xf = new_flat[:, 0, :].reshape(-1)
    Yf = new_flat[:, 1, :].reshape(-1)

    im_flat = im.reshape(-1, C)
    mesh = plsc.VectorSubcoreMesh(core_axis_name="c", subcore_axis_name="s",
                                  num_cores=NC, num_subcores=NS)
    nbr = pl.kernel(
        _build_body,
        out_type=jax.ShapeDtypeStruct((PIX, 4 * C), jnp.float32),
        mesh=mesh,
        scratch_types=[],
        compiler_params=pltpu.CompilerParams(use_tc_tiling_on_sc=False),
    )(im_flat)
    scratch = (
        [pltpu.VMEM((NCB,), jnp.float32) for _ in range(2)]
        + [pltpu.VMEM((N,), jnp.int32) for _ in range(2)]
        + [pltpu.VMEM((N,), jnp.float32) for _ in range(8)]
        + [pltpu.VMEM((N, 4 * C), jnp.float32) for _ in range(2)]
        + [pltpu.VMEM((N * C,), jnp.float32) for _ in range(2)]
        + [pltpu.SemaphoreType.DMA for _ in range(4)]
    )
    out_flat = pl.kernel(
        _body,
        out_type=jax.ShapeDtypeStruct((PIX * C,), jnp.float32),
        mesh=mesh,
        scratch_types=scratch,
        compiler_params=pltpu.CompilerParams(use_tc_tiling_on_sc=False),
    )(nbr, Xf, Yf)
    return out_flat.reshape(B, H, W, C)


# speedup vs baseline: 3.9505x; 3.9505x over previous
import jax
import jax.numpy as jnp
from jax import lax
from jax.experimental import pallas as pl
from jax.experimental.pallas import tpu as pltpu
from jax.experimental.pallas import tpu_sc as plsc

B, H, W, C = 16, 384, 384, 96
HW = H * W
NC, NS, L = 2, 16, 16
NW = NC * NS
PPW = HW // NW
N = 96
CHUNKS = PPW // N
G = N // L
CB = 8
NCB = N * CB
Wf = float(W)
Hf = float(H)

_DNUMS = lax.GatherDimensionNumbers(
    offset_dims=(), collapsed_slice_dims=(0,), start_index_map=(0,))


def _splat(v, l):
    idx = jnp.full((L, 1), l, jnp.int32)
    return lax.gather(v, idx, _DNUMS, (1,),
                      mode=lax.GatherScatterMode.PROMISE_IN_BOUNDS)


def _gen_and_fire(c, pix0, xf_hbm, yf_hbm, nbr_hbm,
                  cbx, cby, idx, ws, gbuf, sem):
    @pl.when(lax.rem(c, CB) == 0)
    def _():
        off = pix0 + c * N
        pltpu.sync_copy(xf_hbm.at[pl.ds(off, NCB)], cbx)
        pltpu.sync_copy(yf_hbm.at[pl.ds(off, NCB)], cby)

    pos = lax.rem(c, CB) * N
    wA, wB, wC, wD = ws
    for g in range(G):
        s = pl.ds(g * L, L)
        Xf = cbx[pl.ds(pos + g * L, L)]
        Yf = cby[pl.ds(pos + g * L, L)]
        Xp = (Xf + 1.0) / 2.0 * Wf
        Yp = (Yf + 1.0) / 2.0 * Hf
        x0t = Xp.astype(jnp.int32)
        x0 = jnp.where(x0t.astype(jnp.float32) > Xp, x0t - 1, x0t)
        y0t = Yp.astype(jnp.int32)
        y0 = jnp.where(y0t.astype(jnp.float32) > Yp, y0t - 1, y0t)
        m = ((Xp >= 0.0) & (Xp < Wf - 1.0)
             & (Yp >= 0.0) & (Yp < Hf - 1.0))
        x0c = jnp.minimum(jnp.maximum(x0, 0), W - 2)
        y0c = jnp.minimum(jnp.maximum(y0, 0), H - 2)
        x0f = x0c.astype(jnp.float32)
        y0f = y0c.astype(jnp.float32)
        x1f = x0f + 1.0
        y1f = y0f + 1.0
        zero = jnp.zeros((L,), jnp.float32)
        wa = jnp.where(m, (x1f - Xp) * (y1f - Yp), zero)
        wb = jnp.where(m, (x1f - Xp) * (Yp - y0f), zero)
        wc = jnp.where(m, (Xp - x0f) * (y1f - Yp), zero)
        wd = jnp.where(m, (Xp - x0f) * (Yp - y0f), zero)
        idx[s] = y0c * W + x0c
        wA[s] = wa
        wB[s] = wb
        wC[s] = wc
        wD[s] = wd
    pltpu.async_copy(nbr_hbm.at[idx], gbuf, sem)


def _combine(ws, gbuf, ob):
    wA, wB, wC, wD = ws

    def gbody(g, carry):
        s = pl.ds(g * L, L)
        wa16 = wA[s]
        wb16 = wB[s]
        wc16 = wC[s]
        wd16 = wD[s]
        for l in range(L):
            i = g * L + l
            wal = _splat(wa16, l)
            wbl = _splat(wb16, l)
            wcl = _splat(wc16, l)
            wdl = _splat(wd16, l)
            for j in range(C // L):
                av = gbuf[i, pl.ds(j * L, L)]
                cv = gbuf[i, pl.ds(C + j * L, L)]
                bv = gbuf[i, pl.ds(2 * C + j * L, L)]
                dv = gbuf[i, pl.ds(3 * C + j * L, L)]
                o = av * wal + bv * wbl
                o = o + cv * wcl
                o = o + dv * wdl
                ob[pl.ds(i * C + j * L, L)] = o
        return carry

    lax.fori_loop(0, G, gbody, 0)


def _body(nbr_hbm, xf_hbm, yf_hbm, out_hbm,
          cbx, cby,
          idx0, idx1,
          wA0, wB0, wC0, wD0, wA1, wB1, wC1, wD1,
          g0, g1, ob0, ob1,
          sem_g0, sem_g1, sem_o0, sem_o1):
    wid = lax.axis_index("c") * NS + lax.axis_index("s")
    pix0 = wid * PPW

    idxs = (idx0, idx1)
    ws = ((wA0, wB0, wC0, wD0), (wA1, wB1, wC1, wD1))
    gbufs = (g0, g1)
    obs = (ob0, ob1)
    sem_gs = (sem_g0, sem_g1)
    sem_os = (sem_o0, sem_o1)

    _gen_and_fire(0, pix0, xf_hbm, yf_hbm, nbr_hbm,
                  cbx, cby, idxs[0], ws[0], gbufs[0], sem_gs[0])

    def outer(i2, carry):
        for d in (0, 1):
            c = i2 * 2 + d
            nd = 1 - d

            @pl.when(c + 1 < CHUNKS)
            def _():
                _gen_and_fire(c + 1, pix0, xf_hbm, yf_hbm, nbr_hbm,
                              cbx, cby, idxs[nd], ws[nd], gbufs[nd],
                              sem_gs[nd])

            pltpu.make_async_copy(nbr_hbm.at[idxs[d]], gbufs[d],
                                  sem_gs[d]).wait()

            @pl.when(c >= 2)
            def _():
                prev = (pix0 + (c - 2) * N) * C
                pltpu.make_async_copy(
                    obs[d], out_hbm.at[pl.ds(prev, N * C)], sem_os[d]).wait()

            _combine(ws[d], gbufs[d], obs[d])
            cur = (pix0 + c * N) * C
            pltpu.async_copy(obs[d], out_hbm.at[pl.ds(cur, N * C)], sem_os[d])
        return carry

    lax.fori_loop(0, CHUNKS // 2, outer, 0)

    for d in (0, 1):
        last = (pix0 + (CHUNKS - 2 + d) * N) * C
        pltpu.make_async_copy(
            obs[d], out_hbm.at[pl.ds(last, N * C)], sem_os[d]).wait()


@jax.jit
def kernel(im, thetas):
    X, Y = jnp.meshgrid(jnp.linspace(-1.0, 1.0, W), jnp.linspace(-1.0, 1.0, H))
    flat_coords = jnp.concatenate(
        [X.reshape(1, -1), Y.reshape(1, -1),
         jnp.ones((1, H * W), dtype=jnp.float32)], axis=0)
    th = thetas.reshape(-1, 2, 3)
    new_flat = jnp.matmul(th, jnp.broadcast_to(flat_coords[None, :, :],
                                               (B, 3, H * W)))
    Xall = new_flat[:, 0, :]
    Yall = new_flat[:, 1, :]

    mesh = plsc.VectorSubcoreMesh(core_axis_name="c", subcore_axis_name="s",
                                  num_cores=NC, num_subcores=NS)
    scratch = (
        [pltpu.VMEM((NCB,), jnp.float32) for _ in range(2)]
        + [pltpu.VMEM((N,), jnp.int32) for _ in range(2)]
        + [pltpu.VMEM((N,), jnp.float32) for _ in range(8)]
        + [pltpu.VMEM((N, 4 * C), jnp.float32) for _ in range(2)]
        + [pltpu.VMEM((N * C,), jnp.float32) for _ in range(2)]
        + [pltpu.SemaphoreType.DMA for _ in range(4)]
    )
    sc_call = pl.kernel(
        _body,
        out_type=jax.ShapeDtypeStruct((HW * C,), jnp.float32),
        mesh=mesh,
        scratch_types=scratch,
    )

    outs = []
    for b in range(B):
        imb = im[b].reshape(HW, C)
        nbr = jnp.concatenate(
            [imb,
             jnp.roll(imb, -1, axis=0),
             jnp.roll(imb, -W, axis=0),
             jnp.roll(imb, -(W + 1), axis=0)], axis=1)
        outs.append(sc_call(nbr, Xall[b], Yall[b]))
    return jnp.stack(outs).reshape(B, H, W, C)


# speedup vs baseline: 6.9529x; 1.7600x over previous
import jax
import jax.numpy as jnp
from jax import lax
from jax.experimental import pallas as pl
from jax.experimental.pallas import tpu as pltpu
from jax.experimental.pallas import tpu_sc as plsc

B, H, W, C = 16, 384, 384, 96
HW = H * W
NC, NS, L = 2, 16, 16
NW = NC * NS
PPW = B * HW // NW
N = 96
CHUNKS = PPW // N
G = N // L
CB = 8
NCB = N * CB
Wf = float(W)
Hf = float(H)

_DNUMS = lax.GatherDimensionNumbers(
    offset_dims=(), collapsed_slice_dims=(0,), start_index_map=(0,))


def _splat(v, l):
    idx = jnp.full((L, 1), l, jnp.int32)
    return lax.gather(v, idx, _DNUMS, (1,),
                      mode=lax.GatherScatterMode.PROMISE_IN_BOUNDS)


def _gen_and_fire(c, pix0, xf_hbm, yf_hbm, nbr_hbm,
                  cbx, cby, idx, ws, gbuf, sem):
    @pl.when(lax.rem(c, CB) == 0)
    def _():
        off = pix0 + c * N
        pltpu.sync_copy(xf_hbm.at[pl.ds(off, NCB)], cbx)
        pltpu.sync_copy(yf_hbm.at[pl.ds(off, NCB)], cby)

    pos = lax.rem(c, CB) * N
    base = lax.div(pix0 + c * N, HW) * HW
    wA, wB, wC, wD = ws
    for g in range(G):
        s = pl.ds(g * L, L)
        Xf = cbx[pl.ds(pos + g * L, L)]
        Yf = cby[pl.ds(pos + g * L, L)]
        Xp = (Xf + 1.0) / 2.0 * Wf
        Yp = (Yf + 1.0) / 2.0 * Hf
        x0t = Xp.astype(jnp.int32)
        x0 = jnp.where(x0t.astype(jnp.float32) > Xp, x0t - 1, x0t)
        y0t = Yp.astype(jnp.int32)
        y0 = jnp.where(y0t.astype(jnp.float32) > Yp, y0t - 1, y0t)
        m = ((Xp >= 0.0) & (Xp < Wf - 1.0)
             & (Yp >= 0.0) & (Yp < Hf - 1.0))
        x0c = jnp.minimum(jnp.maximum(x0, 0), W - 2)
        y0c = jnp.minimum(jnp.maximum(y0, 0), H - 2)
        x0f = x0c.astype(jnp.float32)
        y0f = y0c.astype(jnp.float32)
        x1f = x0f + 1.0
        y1f = y0f + 1.0
        zero = jnp.zeros((L,), jnp.float32)
        wa = jnp.where(m, (x1f - Xp) * (y1f - Yp), zero)
        wb = jnp.where(m, (x1f - Xp) * (Yp - y0f), zero)
        wc = jnp.where(m, (Xp - x0f) * (y1f - Yp), zero)
        wd = jnp.where(m, (Xp - x0f) * (Yp - y0f), zero)
        idx[s] = base + y0c * W + x0c
        wA[s] = wa
        wB[s] = wb
        wC[s] = wc
        wD[s] = wd
    pltpu.async_copy(nbr_hbm.at[idx], gbuf, sem)


def _combine(ws, gbuf, ob):
    wA, wB, wC, wD = ws

    def gbody(g, carry):
        s = pl.ds(g * L, L)
        wa16 = wA[s]
        wb16 = wB[s]
        wc16 = wC[s]
        wd16 = wD[s]
        for l in range(L):
            i = g * L + l
            wal = _splat(wa16, l)
            wbl = _splat(wb16, l)
            wcl = _splat(wc16, l)
            wdl = _splat(wd16, l)
            for j in range(C // L):
                av = gbuf[i, pl.ds(j * L, L)]
                cv = gbuf[i, pl.ds(C + j * L, L)]
                bv = gbuf[i, pl.ds(2 * C + j * L, L)]
                dv = gbuf[i, pl.ds(3 * C + j * L, L)]
                o = av * wal + bv * wbl
                o = o + cv * wcl
                o = o + dv * wdl
                ob[pl.ds(i * C + j * L, L)] = o
        return carry

    lax.fori_loop(0, G, gbody, 0)


def _body(nbr_hbm, xf_hbm, yf_hbm, out_hbm,
          cbx, cby,
          idx0, idx1,
          wA0, wB0, wC0, wD0, wA1, wB1, wC1, wD1,
          g0, g1, ob0, ob1,
          sem_g0, sem_g1, sem_o0, sem_o1):
    wid = lax.axis_index("c") * NS + lax.axis_index("s")
    pix0 = wid * PPW

    idxs = (idx0, idx1)
    ws = ((wA0, wB0, wC0, wD0), (wA1, wB1, wC1, wD1))
    gbufs = (g0, g1)
    obs = (ob0, ob1)
    sem_gs = (sem_g0, sem_g1)
    sem_os = (sem_o0, sem_o1)

    _gen_and_fire(0, pix0, xf_hbm, yf_hbm, nbr_hbm,
                  cbx, cby, idxs[0], ws[0], gbufs[0], sem_gs[0])

    def outer(i2, carry):
        for d in (0, 1):
            c = i2 * 2 + d
            nd = 1 - d

            @pl.when(c + 1 < CHUNKS)
            def _():
                _gen_and_fire(c + 1, pix0, xf_hbm, yf_hbm, nbr_hbm,
                              cbx, cby, idxs[nd], ws[nd], gbufs[nd],
                              sem_gs[nd])

            pltpu.make_async_copy(nbr_hbm.at[idxs[d]], gbufs[d],
                                  sem_gs[d]).wait()

            @pl.when(c >= 2)
            def _():
                prev = (pix0 + (c - 2) * N) * C
                pltpu.make_async_copy(
                    obs[d], out_hbm.at[pl.ds(prev, N * C)], sem_os[d]).wait()

            _combine(ws[d], gbufs[d], obs[d])
            cur = (pix0 + c * N) * C
            pltpu.async_copy(obs[d], out_hbm.at[pl.ds(cur, N * C)], sem_os[d])
        return carry

    lax.fori_loop(0, CHUNKS // 2, outer, 0)

    for d in (0, 1):
        last = (pix0 + (CHUNKS - 2 + d) * N) * C
        pltpu.make_async_copy(
            obs[d], out_hbm.at[pl.ds(last, N * C)], sem_os[d]).wait()


@jax.jit
def kernel(im, thetas):
    X, Y = jnp.meshgrid(jnp.linspace(-1.0, 1.0, W), jnp.linspace(-1.0, 1.0, H))
    flat_coords = jnp.concatenate(
        [X.reshape(1, -1), Y.reshape(1, -1),
         jnp.ones((1, H * W), dtype=jnp.float32)], axis=0)
    th = thetas.reshape(-1, 2, 3)
    new_flat = jnp.matmul(th, jnp.broadcast_to(flat_coords[None, :, :],
                                               (B, 3, H * W)))
    Xall = new_flat[:, 0, :].reshape(-1)
    Yall = new_flat[:, 1, :].reshape(-1)

    mesh = plsc.VectorSubcoreMesh(core_axis_name="c", subcore_axis_name="s",
                                  num_cores=NC, num_subcores=NS)
    scratch = (
        [pltpu.VMEM((NCB,), jnp.float32) for _ in range(2)]
        + [pltpu.VMEM((N,), jnp.int32) for _ in range(2)]
        + [pltpu.VMEM((N,), jnp.float32) for _ in range(8)]
        + [pltpu.VMEM((N, 4 * C), jnp.float32) for _ in range(2)]
        + [pltpu.VMEM((N * C,), jnp.float32) for _ in range(2)]
        + [pltpu.SemaphoreType.DMA for _ in range(4)]
    )
    sc_call = pl.kernel(
        _body,
        out_type=jax.ShapeDtypeStruct((B * HW * C,), jnp.float32),
        mesh=mesh,
        scratch_types=scratch,
    )

    imf = im.reshape(B * HW, C)
    nbr = jnp.concatenate(
        [imf,
         jnp.roll(imf, -1, axis=0),
         jnp.roll(imf, -W, axis=0),
         jnp.roll(imf, -(W + 1), axis=0)], axis=1)
    return sc_call(nbr, Xall, Yall).reshape(B, H, W, C)
